# Initial kernel scaffold; baseline (speedup 1.0000x reference)
import functools

import jax
import jax.numpy as jnp
from jax import lax
from jax.experimental import pallas as pl
from jax.experimental.pallas import tpu as pltpu
from jax.experimental.pallas import tpu_sc as plsc

N_NODES = 10000
N_EDGES = 320000
D_FEAT = 128
N_CLASSES = 16

NC = 2
NS = 16
NW = NC * NS

RPT = 313
OUT_ROWS = NW * RPT
TRASH = RPT
CH = 4000
NCH = N_EDGES // CH
VECS = CH // 16
G = 128
PEND = 160

ROW_BLK = 1000


def _tc_pool_body(x_ref, wpT_ref, bp_ref, wsT_ref, pooled_ref, xs_ref):
    xb = x_ref[...]
    p = jnp.dot(xb, wpT_ref[...], preferred_element_type=jnp.float32)
    pooled_ref[...] = jnp.maximum(p + bp_ref[...], 0.0)
    xs_ref[...] = jnp.dot(xb, wsT_ref[...], preferred_element_type=jnp.float32)


def _tc_pool(x, wpT, bp, wsT):
    grid = (N_NODES // ROW_BLK,)
    return pl.pallas_call(
        _tc_pool_body,
        grid=grid,
        in_specs=[
            pl.BlockSpec((ROW_BLK, D_FEAT), lambda i: (i, 0)),
            pl.BlockSpec((D_FEAT, D_FEAT), lambda i: (0, 0)),
            pl.BlockSpec((1, D_FEAT), lambda i: (0, 0)),
            pl.BlockSpec((D_FEAT, D_FEAT), lambda i: (0, 0)),
        ],
        out_specs=[
            pl.BlockSpec((ROW_BLK, D_FEAT), lambda i: (i, 0)),
            pl.BlockSpec((ROW_BLK, D_FEAT), lambda i: (i, 0)),
        ],
        out_shape=[
            jax.ShapeDtypeStruct((N_NODES, D_FEAT), jnp.float32),
            jax.ShapeDtypeStruct((N_NODES, D_FEAT), jnp.float32),
        ],
    )(x, wpT, bp, wsT)


def _tc_head_body(xs_ref, hn_ref, wnT_ref, b_ref, wlT_ref, bl_ref, out_ref):
    h = xs_ref[...] + jnp.dot(hn_ref[...], wnT_ref[...],
                              preferred_element_type=jnp.float32) + b_ref[...]
    h = jnp.where(h >= 0.0, h, 0.01 * h)
    z = jnp.dot(h, wlT_ref[...], preferred_element_type=jnp.float32) + bl_ref[...]
    out_ref[...] = jax.nn.sigmoid(z)


def _tc_head(xs, hn, wnT, b, wlT, bl):
    grid = (N_NODES // ROW_BLK,)
    return pl.pallas_call(
        _tc_head_body,
        grid=grid,
        in_specs=[
            pl.BlockSpec((ROW_BLK, D_FEAT), lambda i: (i, 0)),
            pl.BlockSpec((ROW_BLK, D_FEAT), lambda i: (i, 0)),
            pl.BlockSpec((D_FEAT, D_FEAT), lambda i: (0, 0)),
            pl.BlockSpec((1, D_FEAT), lambda i: (0, 0)),
            pl.BlockSpec((D_FEAT, N_CLASSES), lambda i: (0, 0)),
            pl.BlockSpec((1, N_CLASSES), lambda i: (0, 0)),
        ],
        out_specs=pl.BlockSpec((ROW_BLK, N_CLASSES), lambda i: (i, 0)),
        out_shape=jax.ShapeDtypeStruct((N_NODES, N_CLASSES), jnp.float32),
    )(xs, hn, wnT, b, wlT, bl)


def _sc_body(pooled_hbm, src_hbm, dst_hbm, out_hbm,
             acc, dst_buf, src_buf, pend_src, pend_ld, rows_buf, cnt_ref, sem):
    wid = lax.axis_index("s") * NC + lax.axis_index("c")
    lo = wid * RPT
    hi = lo + RPT

    zero16 = jnp.zeros((16,), jnp.float32)

    def _zrow(r, _):
        for f in range(8):
            acc[r, pl.ds(16 * f, 16)] = zero16
        return 0
    lax.fori_loop(0, RPT + 1, _zrow, 0)
    cnt_ref[0] = 0

    def _accum_rows(rows_ref, nrows):
        def _one(j, _):
            ld = pend_ld[j]
            for f in range(8):
                sl = pl.ds(16 * f, 16)
                acc[ld, sl] = jnp.maximum(acc[ld, sl], rows_ref[j, sl])
            return 0
        lax.fori_loop(0, nrows, _one, 0)

    def _chunk(ci, _):
        pltpu.sync_copy(dst_hbm.at[pl.ds(ci * CH, CH)], dst_buf)
        pltpu.sync_copy(src_hbm.at[pl.ds(ci * CH, CH)], src_buf)

        def _vec(v, _):
            e0 = v * 16
            d = dst_buf[pl.ds(e0, 16)]
            s = src_buf[pl.ds(e0, 16)]
            m = (d >= lo) & (d < hi)
            c = cnt_ref[0]
            plsc.store_compressed(pend_src.at[pl.ds(c, 16)], s, mask=m)
            plsc.store_compressed(pend_ld.at[pl.ds(c, 16)], d - lo, mask=m)
            cnt_ref[0] = c + jnp.max(plsc.all_reduce_population_count(m))

            @pl.when(cnt_ref[0] >= G)
            def _flush():
                pltpu.async_copy(
                    pooled_hbm.at[pend_src.at[pl.ds(0, G)]], rows_buf, sem
                ).wait()
                rem_s = pend_src[pl.ds(G, 16)]
                rem_l = pend_ld[pl.ds(G, 16)]
                _accum_rows(rows_buf, G)
                pend_src[pl.ds(0, 16)] = rem_s
                pend_ld[pl.ds(0, 16)] = rem_l
                cnt_ref[0] = cnt_ref[0] - G
            return 0

        lax.fori_loop(0, VECS, _vec, 0)
        return 0

    lax.fori_loop(0, NCH, _chunk, 0)

    c = cnt_ref[0]
    pend_src[pl.ds(c, 16)] = jnp.zeros((16,), jnp.int32)
    pend_ld[pl.ds(c, 16)] = jnp.full((16,), TRASH, jnp.int32)
    nv = (c + 15) // 16

    def _drain(i, _):
        pltpu.async_copy(
            pooled_hbm.at[pend_src.at[pl.ds(i * 16, 16)]],
            rows_buf.at[pl.ds(0, 16)], sem
        ).wait()

        def _one(j, _):
            ld = pend_ld[i * 16 + j]
            for f in range(8):
                sl = pl.ds(16 * f, 16)
                acc[ld, sl] = jnp.maximum(acc[ld, sl], rows_buf[j, sl])
            return 0
        lax.fori_loop(0, 16, _one, 0)
        return 0
    lax.fori_loop(0, nv, _drain, 0)

    pltpu.sync_copy(acc.at[pl.ds(0, RPT)], out_hbm.at[pl.ds(lo, RPT)])


@functools.partial(
    pl.kernel,
    out_type=jax.ShapeDtypeStruct((OUT_ROWS, D_FEAT), jnp.float32),
    mesh=plsc.VectorSubcoreMesh(core_axis_name="c", subcore_axis_name="s"),
    scratch_types=[
        pltpu.VMEM((RPT + 1, D_FEAT), jnp.float32),
        pltpu.VMEM((CH,), jnp.int32),
        pltpu.VMEM((CH,), jnp.int32),
        pltpu.VMEM((PEND,), jnp.int32),
        pltpu.VMEM((PEND,), jnp.int32),
        pltpu.VMEM((G, D_FEAT), jnp.float32),
        pltpu.SMEM((1,), jnp.int32),
        pltpu.SemaphoreType.DMA,
    ],
)
def _sc_segmax(pooled_hbm, src_hbm, dst_hbm, out_hbm,
               acc, dst_buf, src_buf, pend_src, pend_ld, rows_buf, cnt_ref, sem):
    _sc_body(pooled_hbm, src_hbm, dst_hbm, out_hbm,
             acc, dst_buf, src_buf, pend_src, pend_ld, rows_buf, cnt_ref, sem)


def kernel(x, edge_index, W_pool, b_pool, W_self, W_neigh, bias, W_lin, b_lin):
    src = edge_index[0]
    dst = edge_index[1]
    pooled, xs = _tc_pool(x, W_pool.T, b_pool.reshape(1, -1), W_self.T)
    hn_pad = _sc_segmax(pooled, src, dst)
    hn = hn_pad[:N_NODES]
    return _tc_head(xs, hn, W_neigh.T, bias.reshape(1, -1),
                    W_lin.T, b_lin.reshape(1, -1))


# speedup vs baseline: 1.6926x; 1.6926x over previous
import functools

import jax
import jax.numpy as jnp
from jax import lax
from jax.experimental import pallas as pl
from jax.experimental.pallas import tpu as pltpu
from jax.experimental.pallas import tpu_sc as plsc

N_NODES = 10000
N_EDGES = 320000
D_FEAT = 128
N_CLASSES = 16

NC = 2
NS = 16
NW = NC * NS

RPT = 320
OUT_ROWS = NW * RPT
TRASH = RPT
CH = 4000
NCH = N_EDGES // CH
VECS = CH // 16
G = 128
PEND = 160

ROW_BLK = 1000


def _tc_pool_body(x_ref, wpT_ref, bp_ref, wsT_ref, pooled_ref, xs_ref):
    xb = x_ref[...]
    p = jnp.dot(xb, wpT_ref[...], preferred_element_type=jnp.float32)
    pooled_ref[...] = jnp.maximum(p + bp_ref[...], 0.0)
    xs_ref[...] = jnp.dot(xb, wsT_ref[...], preferred_element_type=jnp.float32)


def _tc_pool(x, wpT, bp, wsT):
    grid = (N_NODES // ROW_BLK,)
    return pl.pallas_call(
        _tc_pool_body,
        grid=grid,
        in_specs=[
            pl.BlockSpec((ROW_BLK, D_FEAT), lambda i: (i, 0)),
            pl.BlockSpec((D_FEAT, D_FEAT), lambda i: (0, 0)),
            pl.BlockSpec((1, D_FEAT), lambda i: (0, 0)),
            pl.BlockSpec((D_FEAT, D_FEAT), lambda i: (0, 0)),
        ],
        out_specs=[
            pl.BlockSpec((ROW_BLK, D_FEAT), lambda i: (i, 0)),
            pl.BlockSpec((ROW_BLK, D_FEAT), lambda i: (i, 0)),
        ],
        out_shape=[
            jax.ShapeDtypeStruct((N_NODES, D_FEAT), jnp.float32),
            jax.ShapeDtypeStruct((N_NODES, D_FEAT), jnp.float32),
        ],
    )(x, wpT, bp, wsT)


def _tc_head_body(xs_ref, hn_ref, wnT_ref, b_ref, wlT_ref, bl_ref, out_ref):
    h = xs_ref[...] + jnp.dot(hn_ref[...], wnT_ref[...],
                              preferred_element_type=jnp.float32) + b_ref[...]
    h = jnp.where(h >= 0.0, h, 0.01 * h)
    z = jnp.dot(h, wlT_ref[...], preferred_element_type=jnp.float32) + bl_ref[...]
    out_ref[...] = jax.nn.sigmoid(z)


def _tc_head(xs, hn, wnT, b, wlT, bl):
    grid = (N_NODES // ROW_BLK,)
    return pl.pallas_call(
        _tc_head_body,
        grid=grid,
        in_specs=[
            pl.BlockSpec((ROW_BLK, D_FEAT), lambda i: (i, 0)),
            pl.BlockSpec((ROW_BLK, D_FEAT), lambda i: (i, 0)),
            pl.BlockSpec((D_FEAT, D_FEAT), lambda i: (0, 0)),
            pl.BlockSpec((1, D_FEAT), lambda i: (0, 0)),
            pl.BlockSpec((D_FEAT, N_CLASSES), lambda i: (0, 0)),
            pl.BlockSpec((1, N_CLASSES), lambda i: (0, 0)),
        ],
        out_specs=pl.BlockSpec((ROW_BLK, N_CLASSES), lambda i: (i, 0)),
        out_shape=jax.ShapeDtypeStruct((N_NODES, N_CLASSES), jnp.float32),
    )(xs, hn, wnT, b, wlT, bl)


def _sc_body(pooled_hbm, src_hbm, dst_hbm, out_hbm,
             acc, dst_buf, src_buf, pend_src, pend_ld, rows_buf, cnt_ref, sem):
    wid = lax.axis_index("s") * NC + lax.axis_index("c")
    lo = wid * RPT
    hi = lo + RPT

    zero16 = jnp.zeros((16,), jnp.float32)

    def _zrow(r, _):
        for f in range(8):
            acc[r, pl.ds(16 * f, 16)] = zero16
        return 0
    lax.fori_loop(0, RPT + 1, _zrow, 0)
    cnt_ref[0] = 0

    def _accum_group(rows_ref, row_base, pend_base):
        ldv = pend_ld[pl.ds(pend_base, 16)]
        for i in range(16):
            ld = ldv[i]
            for f in range(8):
                sl = pl.ds(16 * f, 16)
                acc[ld, sl] = jnp.maximum(acc[ld, sl], rows_ref[row_base + i, sl])

    def _chunk(ci, _):
        pltpu.sync_copy(dst_hbm.at[pl.ds(ci * CH, CH)], dst_buf)
        pltpu.sync_copy(src_hbm.at[pl.ds(ci * CH, CH)], src_buf)

        def _vec(v, _):
            e0 = v * 16
            d = dst_buf[pl.ds(e0, 16)]
            s = src_buf[pl.ds(e0, 16)]
            m = (d >= lo) & (d < hi)
            c = cnt_ref[0]
            cumv = plsc.cumsum(jnp.where(m, 1, 0))
            pos = c + cumv - 1
            plsc.store_scatter(pend_src, [pos], s, mask=m)
            plsc.store_scatter(pend_ld, [pos], d - lo, mask=m)
            cnt_ref[0] = c + cumv[15]

            @pl.when(cnt_ref[0] >= G)
            def _flush():
                pltpu.async_copy(
                    pooled_hbm.at[pend_src.at[pl.ds(0, G)]], rows_buf, sem
                ).wait()
                rem_s = pend_src[pl.ds(G, 16)]
                rem_l = pend_ld[pl.ds(G, 16)]

                def _agrp(g, _):
                    _accum_group(rows_buf, g * 16, g * 16)
                    return 0
                lax.fori_loop(0, G // 16, _agrp, 0)
                pend_src[pl.ds(0, 16)] = rem_s
                pend_ld[pl.ds(0, 16)] = rem_l
                cnt_ref[0] = cnt_ref[0] - G
            return 0

        lax.fori_loop(0, VECS, _vec, 0)
        return 0

    lax.fori_loop(0, NCH, _chunk, 0)

    c = cnt_ref[0]
    pend_src[pl.ds(c, 16)] = jnp.zeros((16,), jnp.int32)
    pend_ld[pl.ds(c, 16)] = jnp.full((16,), TRASH, jnp.int32)
    nv = (c + 15) // 16

    def _drain(i, _):
        pltpu.async_copy(
            pooled_hbm.at[pend_src.at[pl.ds(i * 16, 16)]],
            rows_buf.at[pl.ds(0, 16)], sem
        ).wait()

        _accum_group(rows_buf, 0, i * 16)
        return 0
    lax.fori_loop(0, nv, _drain, 0)

    pltpu.sync_copy(acc.at[pl.ds(0, RPT)], out_hbm.at[pl.ds(lo, RPT)])


@functools.partial(
    pl.kernel,
    out_type=jax.ShapeDtypeStruct((OUT_ROWS, D_FEAT), jnp.float32),
    mesh=plsc.VectorSubcoreMesh(core_axis_name="c", subcore_axis_name="s"),
    compiler_params=pltpu.CompilerParams(needs_layout_passes=False),
    scratch_types=[
        pltpu.VMEM((RPT + 1, D_FEAT), jnp.float32),
        pltpu.VMEM((CH,), jnp.int32),
        pltpu.VMEM((CH,), jnp.int32),
        pltpu.VMEM((PEND,), jnp.int32),
        pltpu.VMEM((PEND,), jnp.int32),
        pltpu.VMEM((G, D_FEAT), jnp.float32),
        pltpu.SMEM((1,), jnp.int32),
        pltpu.SemaphoreType.DMA,
    ],
)
def _sc_segmax(pooled_hbm, src_hbm, dst_hbm, out_hbm,
               acc, dst_buf, src_buf, pend_src, pend_ld, rows_buf, cnt_ref, sem):
    _sc_body(pooled_hbm, src_hbm, dst_hbm, out_hbm,
             acc, dst_buf, src_buf, pend_src, pend_ld, rows_buf, cnt_ref, sem)


def kernel(x, edge_index, W_pool, b_pool, W_self, W_neigh, bias, W_lin, b_lin):
    src = edge_index[0]
    dst = edge_index[1]
    pooled, xs = _tc_pool(x, W_pool.T, b_pool.reshape(1, -1), W_self.T)
    hn_pad = _sc_segmax(pooled, src, dst)
    hn = hn_pad[:N_NODES]
    return _tc_head(xs, hn, W_neigh.T, bias.reshape(1, -1),
                    W_lin.T, b_lin.reshape(1, -1))


# speedup vs baseline: 1.7869x; 1.0557x over previous
import functools

import jax
import jax.numpy as jnp
from jax import lax
from jax.experimental import pallas as pl
from jax.experimental.pallas import tpu as pltpu
from jax.experimental.pallas import tpu_sc as plsc

N_NODES = 10000
N_EDGES = 320000
D_FEAT = 128
N_CLASSES = 16

NC = 2
NS = 16
NW = NC * NS

RPT = 320
OUT_ROWS = NW * RPT
TRASH = RPT
CH = 4000
NCH = N_EDGES // CH
VECS = CH // 16
G = 128
PEND = 160

ROW_BLK = 1000


def _tc_pool_body(x_ref, wpT_ref, bp_ref, wsT_ref, pooled_ref, xs_ref):
    xb = x_ref[...]
    p = jnp.dot(xb, wpT_ref[...], preferred_element_type=jnp.float32)
    pooled_ref[...] = jnp.maximum(p + bp_ref[...], 0.0)
    xs_ref[...] = jnp.dot(xb, wsT_ref[...], preferred_element_type=jnp.float32)


def _tc_pool(x, wpT, bp, wsT):
    grid = (N_NODES // ROW_BLK,)
    return pl.pallas_call(
        _tc_pool_body,
        grid=grid,
        in_specs=[
            pl.BlockSpec((ROW_BLK, D_FEAT), lambda i: (i, 0)),
            pl.BlockSpec((D_FEAT, D_FEAT), lambda i: (0, 0)),
            pl.BlockSpec((1, D_FEAT), lambda i: (0, 0)),
            pl.BlockSpec((D_FEAT, D_FEAT), lambda i: (0, 0)),
        ],
        out_specs=[
            pl.BlockSpec((ROW_BLK, D_FEAT), lambda i: (i, 0)),
            pl.BlockSpec((ROW_BLK, D_FEAT), lambda i: (i, 0)),
        ],
        out_shape=[
            jax.ShapeDtypeStruct((N_NODES, D_FEAT), jnp.float32),
            jax.ShapeDtypeStruct((N_NODES, D_FEAT), jnp.float32),
        ],
    )(x, wpT, bp, wsT)


def _tc_head_body(xs_ref, hn_ref, wnT_ref, b_ref, wlT_ref, bl_ref, out_ref):
    h = xs_ref[...] + jnp.dot(hn_ref[...], wnT_ref[...],
                              preferred_element_type=jnp.float32) + b_ref[...]
    h = jnp.where(h >= 0.0, h, 0.01 * h)
    z = jnp.dot(h, wlT_ref[...], preferred_element_type=jnp.float32) + bl_ref[...]
    out_ref[...] = jax.nn.sigmoid(z)


def _tc_head(xs, hn, wnT, b, wlT, bl):
    grid = (N_NODES // ROW_BLK,)
    return pl.pallas_call(
        _tc_head_body,
        grid=grid,
        in_specs=[
            pl.BlockSpec((ROW_BLK, D_FEAT), lambda i: (i, 0)),
            pl.BlockSpec((ROW_BLK, D_FEAT), lambda i: (i, 0)),
            pl.BlockSpec((D_FEAT, D_FEAT), lambda i: (0, 0)),
            pl.BlockSpec((1, D_FEAT), lambda i: (0, 0)),
            pl.BlockSpec((D_FEAT, N_CLASSES), lambda i: (0, 0)),
            pl.BlockSpec((1, N_CLASSES), lambda i: (0, 0)),
        ],
        out_specs=pl.BlockSpec((ROW_BLK, N_CLASSES), lambda i: (i, 0)),
        out_shape=jax.ShapeDtypeStruct((N_NODES, N_CLASSES), jnp.float32),
    )(xs, hn, wnT, b, wlT, bl)


def _sc_body(pooled_hbm, src_hbm, dst_hbm, out_hbm,
             acc, dst_buf, src_buf, dst_buf2, src_buf2,
             pend_src, pend_ld, rows_buf, cnt_ref, sem, semd, sems):
    wid = lax.axis_index("s") * NC + lax.axis_index("c")
    lo = wid * RPT
    hi = lo + RPT

    zero16 = jnp.zeros((16,), jnp.float32)

    def _zrow(r, _):
        for f in range(8):
            acc[r, pl.ds(16 * f, 16)] = zero16
        return 0
    lax.fori_loop(0, RPT + 1, _zrow, 0)
    cnt_ref[0] = 0

    def _accum_group(rows_ref, row_base, pend_base):
        ldv = pend_ld[pl.ds(pend_base, 16)]
        for i in range(16):
            ld = ldv[i]
            for f in range(8):
                sl = pl.ds(16 * f, 16)
                acc[ld, sl] = jnp.maximum(acc[ld, sl], rows_ref[row_base + i, sl])

    def _scan_buf(db, sb):
        def _vec(v, _):
            e0 = v * 16
            d = db[pl.ds(e0, 16)]
            m = (d >= lo) & (d < hi)
            pc = plsc.all_reduce_population_count(m)

            @pl.when(pc[0] > 0)
            def _match():
                s = sb[pl.ds(e0, 16)]
                c = cnt_ref[0]
                cumv = plsc.cumsum(jnp.where(m, 1, 0))
                pos = c + cumv - 1
                plsc.store_scatter(pend_src, [pos], s, mask=m)
                plsc.store_scatter(pend_ld, [pos], d - lo, mask=m)
                cnt_ref[0] = c + pc[0]

                @pl.when(cnt_ref[0] >= G)
                def _flush():
                    pltpu.async_copy(
                        pooled_hbm.at[pend_src.at[pl.ds(0, G)]], rows_buf, sem
                    ).wait()
                    rem_s = pend_src[pl.ds(G, 16)]
                    rem_l = pend_ld[pl.ds(G, 16)]

                    def _agrp(g, _):
                        _accum_group(rows_buf, g * 16, g * 16)
                        return 0
                    lax.fori_loop(0, G // 16, _agrp, 0)
                    pend_src[pl.ds(0, 16)] = rem_s
                    pend_ld[pl.ds(0, 16)] = rem_l
                    cnt_ref[0] = cnt_ref[0] - G
            return 0

        lax.fori_loop(0, VECS, _vec, 0)

    def _start_chunk(ci, db, sb):
        pltpu.async_copy(dst_hbm.at[pl.ds(ci * CH, CH)], db, semd)
        pltpu.async_copy(src_hbm.at[pl.ds(ci * CH, CH)], sb, sems)

    def _wait_chunk(ci, db, sb):
        pltpu.make_async_copy(dst_hbm.at[pl.ds(ci * CH, CH)], db, semd).wait()
        pltpu.make_async_copy(src_hbm.at[pl.ds(ci * CH, CH)], sb, sems).wait()

    _start_chunk(0, dst_buf, src_buf)

    def _outer(g, _):
        for u, (db, sb) in ((0, (dst_buf, src_buf)), (1, (dst_buf2, src_buf2))):
            ci = 2 * g + u
            _wait_chunk(ci, db, sb)
            nb = (dst_buf2, src_buf2) if u == 0 else (dst_buf, src_buf)

            @pl.when(ci + 1 < NCH)
            def _pref():
                _start_chunk(ci + 1, nb[0], nb[1])
            _scan_buf(db, sb)
        return 0

    lax.fori_loop(0, NCH // 2, _outer, 0)

    c = cnt_ref[0]
    pend_src[pl.ds(c, 16)] = jnp.zeros((16,), jnp.int32)
    pend_ld[pl.ds(c, 16)] = jnp.full((16,), TRASH, jnp.int32)
    nv = (c + 15) // 16

    def _drain(i, _):
        pltpu.async_copy(
            pooled_hbm.at[pend_src.at[pl.ds(i * 16, 16)]],
            rows_buf.at[pl.ds(0, 16)], sem
        ).wait()

        _accum_group(rows_buf, 0, i * 16)
        return 0
    lax.fori_loop(0, nv, _drain, 0)

    pltpu.sync_copy(acc.at[pl.ds(0, RPT)], out_hbm.at[pl.ds(lo, RPT)])


@functools.partial(
    pl.kernel,
    out_type=jax.ShapeDtypeStruct((OUT_ROWS, D_FEAT), jnp.float32),
    mesh=plsc.VectorSubcoreMesh(core_axis_name="c", subcore_axis_name="s"),
    compiler_params=pltpu.CompilerParams(needs_layout_passes=False),
    scratch_types=[
        pltpu.VMEM((RPT + 1, D_FEAT), jnp.float32),
        pltpu.VMEM((CH,), jnp.int32),
        pltpu.VMEM((CH,), jnp.int32),
        pltpu.VMEM((CH,), jnp.int32),
        pltpu.VMEM((CH,), jnp.int32),
        pltpu.VMEM((PEND,), jnp.int32),
        pltpu.VMEM((PEND,), jnp.int32),
        pltpu.VMEM((G, D_FEAT), jnp.float32),
        pltpu.SMEM((1,), jnp.int32),
        pltpu.SemaphoreType.DMA,
        pltpu.SemaphoreType.DMA,
        pltpu.SemaphoreType.DMA,
    ],
)
def _sc_segmax(pooled_hbm, src_hbm, dst_hbm, out_hbm,
               acc, dst_buf, src_buf, dst_buf2, src_buf2,
               pend_src, pend_ld, rows_buf, cnt_ref, sem, semd, sems):
    _sc_body(pooled_hbm, src_hbm, dst_hbm, out_hbm,
             acc, dst_buf, src_buf, dst_buf2, src_buf2,
             pend_src, pend_ld, rows_buf, cnt_ref, sem, semd, sems)


def kernel(x, edge_index, W_pool, b_pool, W_self, W_neigh, bias, W_lin, b_lin):
    src = edge_index[0]
    dst = edge_index[1]
    pooled, xs = _tc_pool(x, W_pool.T, b_pool.reshape(1, -1), W_self.T)
    hn_pad = _sc_segmax(pooled, src, dst)
    hn = hn_pad[:N_NODES]
    return _tc_head(xs, hn, W_neigh.T, bias.reshape(1, -1),
                    W_lin.T, b_lin.reshape(1, -1))


# speedup vs baseline: 2.4420x; 1.3666x over previous
import functools

import jax
import jax.numpy as jnp
from jax import lax
from jax.experimental import pallas as pl
from jax.experimental.pallas import tpu as pltpu
from jax.experimental.pallas import tpu_sc as plsc

N_NODES = 10000
N_EDGES = 320000
D_FEAT = 128
N_CLASSES = 16

NC = 2
NS = 16
NW = NC * NS

RPT = 320
OUT_ROWS = NW * RPT
TRASH = RPT
CH = 6400
NCH = N_EDGES // CH
VECS = CH // 16
BLK = 8
G = 128
PEND = 272

ROW_BLK = 1000


def _tc_pool_body(x_ref, wpT_ref, bp_ref, wsT_ref, pooled_ref, xs_ref):
    xb = x_ref[...]
    p = jnp.dot(xb, wpT_ref[...], preferred_element_type=jnp.float32)
    pooled_ref[...] = jnp.maximum(p + bp_ref[...], 0.0)
    xs_ref[...] = jnp.dot(xb, wsT_ref[...], preferred_element_type=jnp.float32)


def _tc_pool(x, wpT, bp, wsT):
    grid = (N_NODES // ROW_BLK,)
    return pl.pallas_call(
        _tc_pool_body,
        grid=grid,
        in_specs=[
            pl.BlockSpec((ROW_BLK, D_FEAT), lambda i: (i, 0)),
            pl.BlockSpec((D_FEAT, D_FEAT), lambda i: (0, 0)),
            pl.BlockSpec((1, D_FEAT), lambda i: (0, 0)),
            pl.BlockSpec((D_FEAT, D_FEAT), lambda i: (0, 0)),
        ],
        out_specs=[
            pl.BlockSpec((ROW_BLK, D_FEAT), lambda i: (i, 0)),
            pl.BlockSpec((ROW_BLK, D_FEAT), lambda i: (i, 0)),
        ],
        out_shape=[
            jax.ShapeDtypeStruct((N_NODES, D_FEAT), jnp.float32),
            jax.ShapeDtypeStruct((N_NODES, D_FEAT), jnp.float32),
        ],
    )(x, wpT, bp, wsT)


def _tc_head_body(xs_ref, hn_ref, wnT_ref, b_ref, wlT_ref, bl_ref, out_ref):
    h = xs_ref[...] + jnp.dot(hn_ref[...], wnT_ref[...],
                              preferred_element_type=jnp.float32) + b_ref[...]
    h = jnp.where(h >= 0.0, h, 0.01 * h)
    z = jnp.dot(h, wlT_ref[...], preferred_element_type=jnp.float32) + bl_ref[...]
    out_ref[...] = jax.nn.sigmoid(z)


def _tc_head(xs, hn, wnT, b, wlT, bl):
    grid = (N_NODES // ROW_BLK,)
    return pl.pallas_call(
        _tc_head_body,
        grid=grid,
        in_specs=[
            pl.BlockSpec((ROW_BLK, D_FEAT), lambda i: (i, 0)),
            pl.BlockSpec((ROW_BLK, D_FEAT), lambda i: (i, 0)),
            pl.BlockSpec((D_FEAT, D_FEAT), lambda i: (0, 0)),
            pl.BlockSpec((1, D_FEAT), lambda i: (0, 0)),
            pl.BlockSpec((D_FEAT, N_CLASSES), lambda i: (0, 0)),
            pl.BlockSpec((1, N_CLASSES), lambda i: (0, 0)),
        ],
        out_specs=pl.BlockSpec((ROW_BLK, N_CLASSES), lambda i: (i, 0)),
        out_shape=jax.ShapeDtypeStruct((N_NODES, N_CLASSES), jnp.float32),
    )(xs, hn, wnT, b, wlT, bl)


def _sc_body(pooled_hbm, src_hbm, dst_hbm, out_hbm,
             acc, dst_buf, src_buf, dst_buf2, src_buf2,
             pend_src, pend_ld, rows_buf, cnt_ref, sem, semd, sems):
    wid = lax.axis_index("s") * NC + lax.axis_index("c")
    lo = wid * RPT
    hi = lo + RPT

    zero16 = jnp.zeros((16,), jnp.float32)

    def _zrow(r, _):
        for f in range(8):
            acc[r, pl.ds(16 * f, 16)] = zero16
        return 0
    lax.fori_loop(0, RPT + 1, _zrow, 0)
    cnt_ref[0] = 0

    def _accum_group(rows_ref, row_base, pend_base):
        ldv = pend_ld[pl.ds(pend_base, 16)]
        for i in range(16):
            ld = ldv[i]
            for f in range(8):
                sl = pl.ds(16 * f, 16)
                acc[ld, sl] = jnp.maximum(acc[ld, sl], rows_ref[row_base + i, sl])

    def _flush_pend(cv):
        pltpu.async_copy(
            pooled_hbm.at[pend_src.at[pl.ds(0, G)]], rows_buf, sem
        ).wait()

        def _agrp(g, _):
            _accum_group(rows_buf, g * 16, g * 16)
            return 0
        lax.fori_loop(0, G // 16, _agrp, 0)
        for k in range(BLK):
            pend_src[pl.ds(16 * k, 16)] = pend_src[pl.ds(G + 16 * k, 16)]
            pend_ld[pl.ds(16 * k, 16)] = pend_ld[pl.ds(G + 16 * k, 16)]
        return cv - G

    def _scan_buf(db, sb, cvec):
        def _blk(b, cv):
            for u in range(BLK):
                e0 = (b * BLK + u) * 16
                d = db[pl.ds(e0, 16)]
                s = sb[pl.ds(e0, 16)]
                m = (d >= lo) & (d < hi)
                cumv = plsc.cumsum(jnp.where(m, 1, 0))
                pos = cv + cumv - 1
                plsc.store_scatter(pend_src, [pos], s, mask=m)
                plsc.store_scatter(pend_ld, [pos], d - lo, mask=m)
                cv = cv + plsc.all_reduce_population_count(m)
            return lax.cond(cv[0] >= G, _flush_pend, lambda x: x, cv)

        return lax.fori_loop(0, VECS // BLK, _blk, cvec)

    def _start_chunk(ci, db, sb):
        pltpu.async_copy(dst_hbm.at[pl.ds(ci * CH, CH)], db, semd)
        pltpu.async_copy(src_hbm.at[pl.ds(ci * CH, CH)], sb, sems)

    def _wait_chunk(ci, db, sb):
        pltpu.make_async_copy(dst_hbm.at[pl.ds(ci * CH, CH)], db, semd).wait()
        pltpu.make_async_copy(src_hbm.at[pl.ds(ci * CH, CH)], sb, sems).wait()

    _start_chunk(0, dst_buf, src_buf)

    def _outer(g, cv):
        for u, (db, sb) in ((0, (dst_buf, src_buf)), (1, (dst_buf2, src_buf2))):
            ci = 2 * g + u
            _wait_chunk(ci, db, sb)
            nb = (dst_buf2, src_buf2) if u == 0 else (dst_buf, src_buf)

            @pl.when(ci + 1 < NCH)
            def _pref():
                _start_chunk(ci + 1, nb[0], nb[1])
            cv = _scan_buf(db, sb, cv)
        return cv

    cvec = lax.fori_loop(0, NCH // 2, _outer, jnp.zeros((16,), jnp.int32))

    c = cvec[0]
    pend_src[pl.ds(c, 16)] = jnp.zeros((16,), jnp.int32)
    pend_ld[pl.ds(c, 16)] = jnp.full((16,), TRASH, jnp.int32)
    nv = (c + 15) // 16

    def _drain(i, _):
        pltpu.async_copy(
            pooled_hbm.at[pend_src.at[pl.ds(i * 16, 16)]],
            rows_buf.at[pl.ds(0, 16)], sem
        ).wait()

        _accum_group(rows_buf, 0, i * 16)
        return 0
    lax.fori_loop(0, nv, _drain, 0)

    pltpu.sync_copy(acc.at[pl.ds(0, RPT)], out_hbm.at[pl.ds(lo, RPT)])


@functools.partial(
    pl.kernel,
    out_type=jax.ShapeDtypeStruct((OUT_ROWS, D_FEAT), jnp.float32),
    mesh=plsc.VectorSubcoreMesh(core_axis_name="c", subcore_axis_name="s"),
    compiler_params=pltpu.CompilerParams(needs_layout_passes=False),
    scratch_types=[
        pltpu.VMEM((RPT + 1, D_FEAT), jnp.float32),
        pltpu.VMEM((CH,), jnp.int32),
        pltpu.VMEM((CH,), jnp.int32),
        pltpu.VMEM((CH,), jnp.int32),
        pltpu.VMEM((CH,), jnp.int32),
        pltpu.VMEM((PEND,), jnp.int32),
        pltpu.VMEM((PEND,), jnp.int32),
        pltpu.VMEM((G, D_FEAT), jnp.float32),
        pltpu.SMEM((1,), jnp.int32),
        pltpu.SemaphoreType.DMA,
        pltpu.SemaphoreType.DMA,
        pltpu.SemaphoreType.DMA,
    ],
)
def _sc_segmax(pooled_hbm, src_hbm, dst_hbm, out_hbm,
               acc, dst_buf, src_buf, dst_buf2, src_buf2,
               pend_src, pend_ld, rows_buf, cnt_ref, sem, semd, sems):
    _sc_body(pooled_hbm, src_hbm, dst_hbm, out_hbm,
             acc, dst_buf, src_buf, dst_buf2, src_buf2,
             pend_src, pend_ld, rows_buf, cnt_ref, sem, semd, sems)


def kernel(x, edge_index, W_pool, b_pool, W_self, W_neigh, bias, W_lin, b_lin):
    src = edge_index[0]
    dst = edge_index[1]
    pooled, xs = _tc_pool(x, W_pool.T, b_pool.reshape(1, -1), W_self.T)
    hn_pad = _sc_segmax(pooled, src, dst)
    hn = hn_pad[:N_NODES]
    return _tc_head(xs, hn, W_neigh.T, bias.reshape(1, -1),
                    W_lin.T, b_lin.reshape(1, -1))
